# baseline (speedup 1.0000x reference)
import dataclasses
import functools

import jax
import jax.numpy as jnp
from jax import lax
from jax.experimental import pallas as pl
from jax.experimental.pallas import tpu as pltpu
from jax.experimental.pallas import tpu_sc as plsc

EMBED_DIM = 64
PAD_DIM = 128
NUM_CORES = 2
NUM_SUBCORES = 16
NUM_WORKERS = NUM_CORES * NUM_SUBCORES
CHUNK_B = 256


@functools.partial(jax.jit, static_argnames=("batch", "hist"))
def _sc_lookup(idx_t, table128, batch, hist):
    bpw = batch // NUM_WORKERS
    sub = bpw // CHUNK_B
    n_chunks = hist * sub
    mesh = plsc.VectorSubcoreMesh(core_axis_name="c", subcore_axis_name="s")
    cp = pltpu.CompilerParams()
    if "needs_layout_passes" in pltpu.CompilerParams.__dataclass_fields__:
        cp = dataclasses.replace(cp, needs_layout_passes=False)

    @functools.partial(
        pl.kernel,
        mesh=mesh,
        compiler_params=cp,
        out_type=jax.ShapeDtypeStruct((hist, EMBED_DIM, batch), jnp.float32),
        scratch_types=[
            pltpu.VMEM((CHUNK_B,), jnp.int32),
            pltpu.VMEM((CHUNK_B,), jnp.int32),
            pltpu.VMEM((CHUNK_B, PAD_DIM), jnp.float32),
            pltpu.VMEM((CHUNK_B, PAD_DIM), jnp.float32),
            pltpu.VMEM((EMBED_DIM, CHUNK_B), jnp.float32),
            pltpu.SemaphoreType.DMA,
            pltpu.SemaphoreType.DMA,
        ],
    )
    def k(idx_hbm, tab_hbm, out_hbm, ib0, ib1, rb0, rb1, otb, s0, s1):
        ibufs = (ib0, ib1)
        rbufs = (rb0, rb1)
        sems = (s0, s1)
        wid = lax.axis_index("s") * NUM_CORES + lax.axis_index("c")
        b_base = wid * bpw
        iota = lax.iota(jnp.int32, 16)
        jvecs = [iota + j0 for j0 in range(0, CHUNK_B, 16)]

        def load_and_fire(t, slot):
            h = t // sub
            b0 = b_base + (t % sub) * CHUNK_B
            pltpu.sync_copy(idx_hbm.at[h, pl.ds(b0, CHUNK_B)], ibufs[slot])
            pltpu.async_copy(tab_hbm.at[ibufs[slot]], rbufs[slot], sems[slot])

        load_and_fire(0, 0)

        @pl.loop(0, n_chunks // 2)
        def _(c):
            for b in range(2):
                t = c * 2 + b
                nslot = (b + 1) % 2

                @pl.when(t + 1 < n_chunks)
                def _():
                    load_and_fire(t + 1, nslot)

                rb = rbufs[b]
                pltpu.make_async_copy(tab_hbm.at[ibufs[b]], rb, sems[b]).wait()

                @pl.loop(0, EMBED_DIM)
                def _(d):
                    d16 = jnp.full((16,), 0, jnp.int32) + d
                    for j, jv in enumerate(jvecs):
                        otb[d, pl.ds(j * 16, 16)] = plsc.load_gather(
                            rb, [jv, d16]
                        )

                h = t // sub
                b0 = b_base + (t % sub) * CHUNK_B
                pltpu.sync_copy(otb, out_hbm.at[h, :, pl.ds(b0, CHUNK_B)])

    return k(idx_t, table128)


def kernel(indices, in_embeddings):
    batch, hist = indices.shape
    table128 = jnp.pad(in_embeddings, ((0, 0), (0, PAD_DIM - EMBED_DIM)))
    out3 = _sc_lookup(indices.T, table128, batch, hist)
    return jnp.transpose(out3, (2, 0, 1))


# speedup vs baseline: 1.3218x; 1.3218x over previous
import dataclasses
import functools

import jax
import jax.numpy as jnp
from jax import lax
from jax.experimental import pallas as pl
from jax.experimental.pallas import tpu as pltpu
from jax.experimental.pallas import tpu_sc as plsc

EMBED_DIM = 64
PAD_DIM = 128
NUM_CORES = 2
NUM_SUBCORES = 16
NUM_WORKERS = NUM_CORES * NUM_SUBCORES
CHUNK = 256


@functools.partial(jax.jit, static_argnames=("total",))
def _sc_gather(flat_idx, table128, total):
    b_per_w = total // NUM_WORKERS
    n_chunks = b_per_w // CHUNK
    mesh = plsc.VectorSubcoreMesh(core_axis_name="c", subcore_axis_name="s")
    cp = pltpu.CompilerParams()
    if "needs_layout_passes" in pltpu.CompilerParams.__dataclass_fields__:
        cp = dataclasses.replace(cp, needs_layout_passes=False)

    @functools.partial(
        pl.kernel,
        mesh=mesh,
        compiler_params=cp,
        out_type=jax.ShapeDtypeStruct((total, PAD_DIM), jnp.float32),
        scratch_types=[
            pltpu.VMEM((b_per_w,), jnp.int32),
            pltpu.VMEM((CHUNK, PAD_DIM), jnp.float32),
            pltpu.VMEM((CHUNK, PAD_DIM), jnp.float32),
            pltpu.SemaphoreType.DMA,
            pltpu.SemaphoreType.DMA,
        ],
    )
    def k(idx_hbm, tab_hbm, out_hbm, idx_v, rb0, rb1, s0, s1):
        rbufs = (rb0, rb1)
        sems = (s0, s1)
        wid = lax.axis_index("s") * NUM_CORES + lax.axis_index("c")
        base = wid * b_per_w
        pltpu.sync_copy(idx_hbm.at[pl.ds(base, b_per_w)], idx_v)

        def fire(t, slot):
            pltpu.async_copy(
                tab_hbm.at[idx_v.at[pl.ds(t * CHUNK, CHUNK)]],
                rbufs[slot],
                sems[slot],
            )

        fire(0, 0)

        @pl.loop(0, n_chunks // 2)
        def _(c):
            for b in range(2):
                t = c * 2 + b
                nslot = (b + 1) % 2

                @pl.when(t + 1 < n_chunks)
                def _():
                    fire(t + 1, nslot)

                rb = rbufs[b]
                pltpu.make_async_copy(
                    tab_hbm.at[idx_v.at[pl.ds(t * CHUNK, CHUNK)]],
                    rb,
                    sems[b],
                ).wait()
                pltpu.sync_copy(rb, out_hbm.at[pl.ds(base + t * CHUNK, CHUNK)])

    return k(flat_idx, table128)


def kernel(indices, in_embeddings):
    batch, hist = indices.shape
    total = batch * hist
    table128 = jnp.pad(in_embeddings, ((0, 0), (0, PAD_DIM - EMBED_DIM)))
    flat_idx = indices.reshape(total)
    out = _sc_gather(flat_idx, table128, total)
    out3 = out.reshape(batch, hist, PAD_DIM)
    return out3[:, :, :EMBED_DIM]
